# baseline (speedup 1.0000x reference)
import jax
import jax.numpy as jnp
from jax.experimental import pallas as pl
from jax.experimental.pallas import tpu as pltpu
from jax.experimental.pallas import tpu_sc as plsc

_B, _S = 4096, 50
_D = 256
_BK = 4
_B_SC = 1024
_B_TC = _B - _B_SC
_TBK = 32


def _lane_bcast(vec, k):
  return jax.lax.gather(
      vec,
      jnp.full((16, 1), k, jnp.int32),
      jax.lax.GatherDimensionNumbers(
          offset_dims=(), collapsed_slice_dims=(0,), start_index_map=(0,)
      ),
      slice_sizes=(1,),
      mode=jax.lax.GatherScatterMode.PROMISE_IN_BOUNDS,
  )


def _sc_lookup(table, x_sc):
  b_sc = x_sc.shape[0]
  vector_mesh = plsc.VectorSubcoreMesh(
      core_axis_name="core", subcore_axis_name="subcore"
  )

  @pl.kernel(
      out_type=jax.ShapeDtypeStruct((b_sc, _S, _D), table.dtype),
      mesh=vector_mesh,
      scratch_types=[pltpu.VMEM((3, _D), jnp.float32)],
      compiler_params=pltpu.CompilerParams(
          use_tc_tiling_on_sc=True, needs_layout_passes=False
      ),
  )
  def sc_kernel(table_hbm, i_hbm, o_hbm, tab_vmem):
    pltpu.sync_copy(table_hbm, tab_vmem)

    def body(i_vmem, o_vmem):
      for h in range(2):
        rows = [
            [tab_vmem[r, pl.ds(h * 128 + g * 16, 16)] for g in range(8)]
            for r in range(3)
        ]

        @pl.loop(0, _BK)
        def _(a):
          for s0 in (0, 16, 32, 34):
            tv = i_vmem[a, pl.ds(s0, 16)]
            for k in range(16):
              rv = _lane_bcast(tv, k)
              m1 = rv == 1
              m2 = rv == 2
              for g in range(8):
                val = jnp.where(
                    m2, rows[2][g], jnp.where(m1, rows[1][g], rows[0][g])
                )
                o_vmem[a, s0 + k, pl.ds(h * 128 + g * 16, 16)] = val

    pltpu.emit_pipeline(
        body,
        grid=(b_sc // _BK,),
        in_specs=[pl.BlockSpec((_BK, _S), index_map=lambda i: (i, 0))],
        out_specs=[pl.BlockSpec((_BK, _S, _D), index_map=lambda i: (i, 0, 0))],
        core_axis_name=("core", "subcore"),
        dimension_semantics=(pltpu.PARALLEL,),
    )(i_hbm, o_hbm)

  return sc_kernel(table, x_sc)


def _tc_body(x_ref, tab_ref, o_ref):
  xb = x_ref[...]
  xb3 = xb[:, :, None]
  t0 = tab_ref[0, :][None, None, :]
  t1 = tab_ref[1, :][None, None, :]
  t2 = tab_ref[2, :][None, None, :]
  o_ref[...] = jnp.where(xb3 == 2, t2, jnp.where(xb3 == 1, t1, t0))


def _tc_lookup(table, x_tc):
  return pl.pallas_call(
      _tc_body,
      grid=(_B_TC // _TBK,),
      in_specs=[
          pl.BlockSpec((_TBK, _S), lambda i: (i, 0)),
          pl.BlockSpec((3, _D), lambda i: (0, 0)),
      ],
      out_specs=pl.BlockSpec((_TBK, _S, _D), lambda i: (i, 0, 0)),
      out_shape=jax.ShapeDtypeStruct((_B, _S, _D), table.dtype),
  )(x_tc, table)


@jax.jit
def kernel(x, table):
  xi = x.astype(jnp.int32)
  sc_out = _sc_lookup(table, xi[_B_TC:])
  tc_out = _tc_lookup(table, xi[:_B_TC])
  return jax.lax.dynamic_update_slice(tc_out, sc_out, (_B_TC, 0, 0))


# speedup vs baseline: 1.1665x; 1.1665x over previous
import jax
import jax.numpy as jnp
from jax.experimental import pallas as pl
from jax.experimental.pallas import tpu as pltpu
from jax.experimental.pallas import tpu_sc as plsc

_B, _S = 4096, 50
_D = 256
_BK = 1


def _lane_bcast(vec, k):
  return jax.lax.gather(
      vec,
      jnp.full((16, 1), k, jnp.int32),
      jax.lax.GatherDimensionNumbers(
          offset_dims=(), collapsed_slice_dims=(0,), start_index_map=(0,)
      ),
      slice_sizes=(1,),
      mode=jax.lax.GatherScatterMode.PROMISE_IN_BOUNDS,
  )


def _sc_lookup(table, x):
  vector_mesh = plsc.VectorSubcoreMesh(
      core_axis_name="core", subcore_axis_name="subcore"
  )

  @pl.kernel(
      out_type=jax.ShapeDtypeStruct((_B, _S, _D), table.dtype),
      mesh=vector_mesh,
      scratch_types=[pltpu.VMEM((3, _D), jnp.float32)],
      compiler_params=pltpu.CompilerParams(
          use_tc_tiling_on_sc=True, needs_layout_passes=False
      ),
  )
  def kernel(table_hbm, i_hbm, o_hbm, tab_vmem):
    pltpu.sync_copy(table_hbm, tab_vmem)

    def body(i_vmem, o_vmem):
      for h in range(2):
        rows = [
            [tab_vmem[r, pl.ds(h * 128 + g * 16, 16)] for g in range(8)]
            for r in range(3)
        ]

        @pl.loop(0, _BK)
        def _(a):
          for s0 in (0, 16, 32, 34):
            tv = i_vmem[a, pl.ds(s0, 16)]
            for k in range(16):
              rv = _lane_bcast(tv, k)
              m1 = rv == 1
              m2 = rv == 2
              for g in range(8):
                val = jnp.where(m2, rows[2][g], jnp.where(m1, rows[1][g], rows[0][g]))
                o_vmem[a, s0 + k, pl.ds(h * 128 + g * 16, 16)] = val

    pltpu.emit_pipeline(
        body,
        grid=(_B // _BK,),
        in_specs=[pl.BlockSpec((_BK, _S), index_map=lambda i: (i, 0))],
        out_specs=[pl.BlockSpec((_BK, _S, _D), index_map=lambda i: (i, 0, 0))],
        core_axis_name=("core", "subcore"),
        dimension_semantics=(pltpu.PARALLEL,),
    )(i_hbm, o_hbm)

  return kernel(table, x)


@jax.jit
def kernel(x, table):
  return _sc_lookup(table, x.astype(jnp.int32))


# speedup vs baseline: 1.1919x; 1.0218x over previous
import jax
import jax.numpy as jnp
from jax.experimental import pallas as pl
from jax.experimental.pallas import tpu as pltpu
from jax.experimental.pallas import tpu_sc as plsc

_B, _S = 4096, 50
_D = 256
_BK = 4


def _lane_bcast(vec, k):
  return jax.lax.gather(
      vec,
      jnp.full((16, 1), k, jnp.int32),
      jax.lax.GatherDimensionNumbers(
          offset_dims=(), collapsed_slice_dims=(0,), start_index_map=(0,)
      ),
      slice_sizes=(1,),
      mode=jax.lax.GatherScatterMode.PROMISE_IN_BOUNDS,
  )


def _sc_lookup(table, x):
  vector_mesh = plsc.VectorSubcoreMesh(
      core_axis_name="core", subcore_axis_name="subcore"
  )

  @pl.kernel(
      out_type=jax.ShapeDtypeStruct((_B, _S, _D), table.dtype),
      mesh=vector_mesh,
      scratch_types=[pltpu.VMEM((3, _D), jnp.float32)],
      compiler_params=pltpu.CompilerParams(
          use_tc_tiling_on_sc=True, needs_layout_passes=False
      ),
  )
  def kernel(table_hbm, i_hbm, o_hbm, tab_vmem):
    pltpu.sync_copy(table_hbm, tab_vmem)

    def body(i_vmem, o_vmem):
      for h in range(2):
        rows = [
            [tab_vmem[r, pl.ds(h * 128 + g * 16, 16)] for g in range(8)]
            for r in range(3)
        ]

        @pl.loop(0, _BK)
        def _(a):
          for s0 in (0, 16, 32, 34):
            tv = i_vmem[a, pl.ds(s0, 16)]
            for k in range(16):
              rv = _lane_bcast(tv, k)
              m1 = rv == 1
              m2 = rv == 2
              for g in range(8):
                val = jnp.where(m2, rows[2][g], jnp.where(m1, rows[1][g], rows[0][g]))
                o_vmem[a, s0 + k, pl.ds(h * 128 + g * 16, 16)] = val

    pltpu.emit_pipeline(
        body,
        grid=(_B // _BK,),
        in_specs=[pl.BlockSpec((_BK, _S), index_map=lambda i: (i, 0))],
        out_specs=[pl.BlockSpec((_BK, _S, _D), index_map=lambda i: (i, 0, 0))],
        core_axis_name=("core", "subcore"),
        dimension_semantics=(pltpu.PARALLEL,),
    )(i_hbm, o_hbm)

  return kernel(table, x)


@jax.jit
def kernel(x, table):
  return _sc_lookup(table, x.astype(jnp.int32))
